# baseline (speedup 1.0000x reference)
import functools

import jax
import jax.numpy as jnp
from jax import lax
from jax.experimental import pallas as pl
from jax.experimental.pallas import tpu as pltpu
from jax.experimental.pallas import tpu_sc as plsc

_N = 10000
_E = 320000
_D = 128
_BN_EPS = 1e-5

_K = 64
_NCHUNK = _E // _K
_NW = 32
_STEPS = -(-_NCHUNK // _NW)
_RPT = 624
_ZR = 104


def _sc_aggr_body(x_hbm, src_hbm, dst_hbm, ea_hbm, out_hbm,
                  sidx0, didx0, ebuf0, xbuf0, sidx1, didx1, ebuf1, xbuf1,
                  aggr, semi0, semi1, seme0, seme1, semx0, semx1):
    c = lax.axis_index("c")
    s = lax.axis_index("s")
    g = s * 2 + c

    sidx = (sidx0, sidx1)
    didx = (didx0, didx1)
    ebuf = (ebuf0, ebuf1)
    xbuf = (xbuf0, xbuf1)
    semi = (semi0, semi1)
    seme = (seme0, seme1)
    semx = (semx0, semx1)

    zv = jnp.zeros((16,), jnp.float32)

    def zrow(r, carry):
        for cc in range(8):
            ebuf0[r, pl.ds(cc * 16, 16)] = zv
        return carry

    lax.fori_loop(0, _K, zrow, 0)
    _nfull = _RPT // _K
    _tail = _RPT - _nfull * _K
    for i in range(_nfull):
        pltpu.sync_copy(ebuf0, aggr.at[pl.ds(s * _RPT + i * _K, _K), :])
    if _tail:
        pltpu.sync_copy(ebuf0.at[pl.ds(0, _tail), :],
                        aggr.at[pl.ds(s * _RPT + _nfull * _K, _tail), :])

    @pl.when(s == 15)
    def _():
        pltpu.sync_copy(ebuf0.at[pl.ds(0, 16), :],
                        aggr.at[pl.ds(16 * _RPT, 16), :])

    plsc.subcore_barrier()

    nact = jnp.where(g < _NCHUNK - (_STEPS - 1) * _NW, _STEPS, _STEPS - 1)

    def issue_idx(t, p):
        base = (g + t * _NW) * _K
        pltpu.async_copy(src_hbm.at[pl.ds(base, _K)], sidx[p], semi[p])
        pltpu.async_copy(dst_hbm.at[pl.ds(base, _K)], didx[p], semi[p])

    def wait_idx(p):
        pltpu.make_async_copy(src_hbm.at[pl.ds(0, _K)], sidx[p], semi[p]).wait()
        pltpu.make_async_copy(dst_hbm.at[pl.ds(0, _K)], didx[p], semi[p]).wait()

    def issue_big(t, p):
        base = (g + t * _NW) * _K
        pltpu.async_copy(ea_hbm.at[pl.ds(base, _K), :], ebuf[p], seme[p])
        pltpu.async_copy(x_hbm.at[sidx[p]], xbuf[p], semx[p])

    def wait_big(p):
        pltpu.make_async_copy(ea_hbm.at[pl.ds(0, _K), :], ebuf[p], seme[p]).wait()
        pltpu.make_async_copy(x_hbm.at[sidx[p]], xbuf[p], semx[p]).wait()

    @pl.when(0 < nact)
    def _():
        issue_idx(0, 0)

    @pl.when(1 < nact)
    def _():
        issue_idx(1, 1)

    @pl.when(0 < nact)
    def _():
        wait_idx(0)
        issue_big(0, 0)

    def stage(t, p):
        @pl.when(t < nact)
        def _():
            @pl.when(t + 1 < nact)
            def _():
                wait_idx(1 - p)
                issue_big(t + 1, 1 - p)

            wait_big(p)
            eb = ebuf[p]
            xb = xbuf[p]

            @plsc.parallel_loop(0, _K, unroll=4)
            def crow(r):
                for cc in range(8):
                    sl = pl.ds(cc * 16, 16)
                    eb[r, sl] = jnp.maximum(eb[r, sl] + xb[r, sl], 0.0)

            pltpu.sync_copy(eb, aggr.at[didx[p]], add=True)

            @pl.when(t + 2 < nact)
            def _():
                issue_idx(t + 2, p)

        return t

    def pair(u, carry):
        stage(2 * u, 0)
        stage(2 * u + 1, 1)
        return carry

    lax.fori_loop(0, (_STEPS + 1) // 2, pair, 0)

    plsc.subcore_barrier()

    for i in range(_RPT // _ZR):
        sl = pl.ds(s * _RPT + i * _ZR, _ZR)
        pltpu.sync_copy(aggr.at[sl, :], out_hbm.at[c, sl, :])

    @pl.when(s == 15)
    def _():
        sl = pl.ds(16 * _RPT, 16)
        pltpu.sync_copy(aggr.at[sl, :], out_hbm.at[c, sl, :])


@functools.cache
def _sc_aggr():
    return pl.kernel(
        _sc_aggr_body,
        mesh=plsc.VectorSubcoreMesh(core_axis_name="c", subcore_axis_name="s"),
        out_type=jax.ShapeDtypeStruct((2, _N, _D), jnp.float32),
        scratch_types=[
            pltpu.VMEM((_K,), jnp.int32),
            pltpu.VMEM((_K,), jnp.int32),
            pltpu.VMEM((_K, _D), jnp.float32),
            pltpu.VMEM((_K, _D), jnp.float32),
            pltpu.VMEM((_K,), jnp.int32),
            pltpu.VMEM((_K,), jnp.int32),
            pltpu.VMEM((_K, _D), jnp.float32),
            pltpu.VMEM((_K, _D), jnp.float32),
            pltpu.VMEM_SHARED((_N, _D), jnp.float32),
            pltpu.SemaphoreType.DMA,
            pltpu.SemaphoreType.DMA,
            pltpu.SemaphoreType.DMA,
            pltpu.SemaphoreType.DMA,
            pltpu.SemaphoreType.DMA,
            pltpu.SemaphoreType.DMA,
        ],
    )


def _tc_mlp(x_ref, p_ref, w1_ref, b1_ref, g1_ref, be1_ref,
            w2_ref, b2_ref, eps_ref, g2_ref, be2_ref, o_ref):
    x = x_ref[...]
    a = p_ref[0] + p_ref[1]
    h = (1.0 + eps_ref[0]) * x + a
    z = jnp.dot(h, w1_ref[...], preferred_element_type=jnp.float32) + b1_ref[...]
    mu = jnp.mean(z, axis=0, keepdims=True)
    d = z - mu
    var = jnp.mean(d * d, axis=0, keepdims=True)
    z = g1_ref[...] * d * lax.rsqrt(var + _BN_EPS) + be1_ref[...]
    z = jnp.maximum(z, 0.0)
    y = x + jnp.dot(z, w2_ref[...], preferred_element_type=jnp.float32) + b2_ref[...]
    mu2 = jnp.mean(y, axis=0, keepdims=True)
    d2 = y - mu2
    var2 = jnp.mean(d2 * d2, axis=0, keepdims=True)
    o_ref[...] = jnp.maximum(
        g2_ref[...] * d2 * lax.rsqrt(var2 + _BN_EPS) + be2_ref[...], 0.0)


def kernel(x, edge_index, edge_attr_emb, W1, b1, bn1_gamma, bn1_beta,
           W2, b2, gine_eps, bn2_gamma, bn2_beta):
    src = edge_index[0]
    dst = edge_index[1]
    partials = _sc_aggr()(x, src, dst, edge_attr_emb)

    out = pl.pallas_call(
        _tc_mlp,
        out_shape=jax.ShapeDtypeStruct((_N, _D), jnp.float32),
        in_specs=[
            pl.BlockSpec(memory_space=pltpu.VMEM),
            pl.BlockSpec(memory_space=pltpu.VMEM),
            pl.BlockSpec(memory_space=pltpu.VMEM),
            pl.BlockSpec(memory_space=pltpu.VMEM),
            pl.BlockSpec(memory_space=pltpu.VMEM),
            pl.BlockSpec(memory_space=pltpu.VMEM),
            pl.BlockSpec(memory_space=pltpu.VMEM),
            pl.BlockSpec(memory_space=pltpu.VMEM),
            pl.BlockSpec(memory_space=pltpu.SMEM),
            pl.BlockSpec(memory_space=pltpu.VMEM),
            pl.BlockSpec(memory_space=pltpu.VMEM),
        ],
        out_specs=pl.BlockSpec(memory_space=pltpu.VMEM),
    )(
        x, partials, W1,
        b1.reshape(1, _D), bn1_gamma.reshape(1, _D), bn1_beta.reshape(1, _D),
        W2, b2.reshape(1, _D), gine_eps.reshape(1),
        bn2_gamma.reshape(1, _D), bn2_beta.reshape(1, _D),
    )
    return out


# speedup vs baseline: 1.1859x; 1.1859x over previous
import functools

import jax
import jax.numpy as jnp
from jax import lax
from jax.experimental import pallas as pl
from jax.experimental.pallas import tpu as pltpu
from jax.experimental.pallas import tpu_sc as plsc

_N = 10000
_E = 320000
_D = 128
_BN_EPS = 1e-5

_K = 64
_NCHUNK = _E // _K
_NW = 32
_STEPS = -(-_NCHUNK // _NW)
_RPT = 624
_ZR = 104


def _sc_aggr_body(x_hbm, src_hbm, dst_hbm, ea_hbm, out_hbm,
                  sidx0, sidx1, didx0, didx1, didx2, didx3,
                  ebuf0, ebuf1, ebuf2, xbuf0, xbuf1,
                  aggr, semi0, semi1, seme0, seme1, seme2,
                  semx0, semx1, sems0, sems1, sems2):
    c = lax.axis_index("c")
    s = lax.axis_index("s")
    g = s * 2 + c

    sidx = (sidx0, sidx1)
    didx = (didx0, didx1, didx2, didx3)
    ebuf = (ebuf0, ebuf1, ebuf2)
    xbuf = (xbuf0, xbuf1)
    semi = (semi0, semi1)
    seme = (seme0, seme1, seme2)
    semx = (semx0, semx1)
    sems = (sems0, sems1, sems2)

    zv = jnp.zeros((16,), jnp.float32)

    def zrow(r, carry):
        for cc in range(8):
            ebuf0[r, pl.ds(cc * 16, 16)] = zv
        return carry

    lax.fori_loop(0, _K, zrow, 0)
    _nfull = _RPT // _K
    _tail = _RPT - _nfull * _K
    for i in range(_nfull):
        pltpu.sync_copy(ebuf0, aggr.at[pl.ds(s * _RPT + i * _K, _K), :])
    if _tail:
        pltpu.sync_copy(ebuf0.at[pl.ds(0, _tail), :],
                        aggr.at[pl.ds(s * _RPT + _nfull * _K, _tail), :])

    @pl.when(s == 15)
    def _():
        pltpu.sync_copy(ebuf0.at[pl.ds(0, 16), :],
                        aggr.at[pl.ds(16 * _RPT, 16), :])

    plsc.subcore_barrier()

    nact = jnp.where(g < _NCHUNK - (_STEPS - 1) * _NW, _STEPS, _STEPS - 1)

    def issue_idx(t, pi, pd):
        base = (g + t * _NW) * _K
        pltpu.async_copy(src_hbm.at[pl.ds(base, _K)], sidx[pi], semi[pi])
        pltpu.async_copy(dst_hbm.at[pl.ds(base, _K)], didx[pd], semi[pi])

    def wait_idx(pi, pd):
        pltpu.make_async_copy(src_hbm.at[pl.ds(0, _K)], sidx[pi], semi[pi]).wait()
        pltpu.make_async_copy(dst_hbm.at[pl.ds(0, _K)], didx[pd], semi[pi]).wait()

    def issue_big(t, px, pe):
        base = (g + t * _NW) * _K
        pltpu.async_copy(ea_hbm.at[pl.ds(base, _K), :], ebuf[pe], seme[pe])
        pltpu.async_copy(x_hbm.at[sidx[px]], xbuf[px], semx[px])

    def wait_big(px, pe):
        pltpu.make_async_copy(ea_hbm.at[pl.ds(0, _K), :], ebuf[pe], seme[pe]).wait()
        pltpu.make_async_copy(x_hbm.at[sidx[px]], xbuf[px], semx[px]).wait()

    def wait_scatter(pe, pd):
        pltpu.make_async_copy(ebuf[pe], aggr.at[didx[pd]], sems[pe]).wait()

    @pl.when(0 < nact)
    def _():
        issue_idx(0, 0, 0)

    @pl.when(1 < nact)
    def _():
        issue_idx(1, 1, 1)

    @pl.when(0 < nact)
    def _():
        wait_idx(0, 0)
        issue_big(0, 0, 0)

    def stage(u, j):
        t = 12 * u + j
        px, pe, pd = j % 2, j % 3, j % 4

        @pl.when(jnp.logical_and(t >= 2, t < nact + 2))
        def _():
            wait_scatter((j - 2) % 3, (j - 2) % 4)

        @pl.when(t + 1 < nact)
        def _():
            wait_idx((j + 1) % 2, (j + 1) % 4)
            issue_big(t + 1, (j + 1) % 2, (j + 1) % 3)

        @pl.when(t < nact)
        def _():
            wait_big(px, pe)

            @pl.when(t + 2 < nact)
            def _():
                issue_idx(t + 2, (j + 2) % 2, (j + 2) % 4)

            eb = ebuf[pe]
            xb = xbuf[px]

            @plsc.parallel_loop(0, _K, unroll=4)
            def crow(r):
                for cc in range(8):
                    sl = pl.ds(cc * 16, 16)
                    eb[r, sl] = jnp.maximum(eb[r, sl] + xb[r, sl], 0.0)

            pltpu.async_copy(eb, aggr.at[didx[pd]], sems[pe], add=True)

    def group(u, carry):
        for j in range(12):
            stage(u, j)
        return carry

    lax.fori_loop(0, -(-(_STEPS + 2) // 12), group, 0)

    plsc.subcore_barrier()

    for i in range(_RPT // _ZR):
        sl = pl.ds(s * _RPT + i * _ZR, _ZR)
        pltpu.sync_copy(aggr.at[sl, :], out_hbm.at[c, sl, :])

    @pl.when(s == 15)
    def _():
        sl = pl.ds(16 * _RPT, 16)
        pltpu.sync_copy(aggr.at[sl, :], out_hbm.at[c, sl, :])


@functools.cache
def _sc_aggr():
    return pl.kernel(
        _sc_aggr_body,
        mesh=plsc.VectorSubcoreMesh(core_axis_name="c", subcore_axis_name="s"),
        out_type=jax.ShapeDtypeStruct((2, _N, _D), jnp.float32),
        scratch_types=(
            [pltpu.VMEM((_K,), jnp.int32)] * 6
            + [pltpu.VMEM((_K, _D), jnp.float32)] * 5
            + [pltpu.VMEM_SHARED((_N, _D), jnp.float32)]
            + [pltpu.SemaphoreType.DMA] * 10
        ),
    )


def _tc_mlp(x_ref, p_ref, w1_ref, b1_ref, g1_ref, be1_ref,
            w2_ref, b2_ref, eps_ref, g2_ref, be2_ref, o_ref):
    x = x_ref[...]
    a = p_ref[0] + p_ref[1]
    h = (1.0 + eps_ref[0]) * x + a
    z = jnp.dot(h, w1_ref[...], preferred_element_type=jnp.float32) + b1_ref[...]
    mu = jnp.mean(z, axis=0, keepdims=True)
    d = z - mu
    var = jnp.mean(d * d, axis=0, keepdims=True)
    z = g1_ref[...] * d * lax.rsqrt(var + _BN_EPS) + be1_ref[...]
    z = jnp.maximum(z, 0.0)
    y = x + jnp.dot(z, w2_ref[...], preferred_element_type=jnp.float32) + b2_ref[...]
    mu2 = jnp.mean(y, axis=0, keepdims=True)
    d2 = y - mu2
    var2 = jnp.mean(d2 * d2, axis=0, keepdims=True)
    o_ref[...] = jnp.maximum(
        g2_ref[...] * d2 * lax.rsqrt(var2 + _BN_EPS) + be2_ref[...], 0.0)


def kernel(x, edge_index, edge_attr_emb, W1, b1, bn1_gamma, bn1_beta,
           W2, b2, gine_eps, bn2_gamma, bn2_beta):
    src = edge_index[0]
    dst = edge_index[1]
    partials = _sc_aggr()(x, src, dst, edge_attr_emb)

    out = pl.pallas_call(
        _tc_mlp,
        out_shape=jax.ShapeDtypeStruct((_N, _D), jnp.float32),
        in_specs=[
            pl.BlockSpec(memory_space=pltpu.VMEM),
            pl.BlockSpec(memory_space=pltpu.VMEM),
            pl.BlockSpec(memory_space=pltpu.VMEM),
            pl.BlockSpec(memory_space=pltpu.VMEM),
            pl.BlockSpec(memory_space=pltpu.VMEM),
            pl.BlockSpec(memory_space=pltpu.VMEM),
            pl.BlockSpec(memory_space=pltpu.VMEM),
            pl.BlockSpec(memory_space=pltpu.VMEM),
            pl.BlockSpec(memory_space=pltpu.SMEM),
            pl.BlockSpec(memory_space=pltpu.VMEM),
            pl.BlockSpec(memory_space=pltpu.VMEM),
        ],
        out_specs=pl.BlockSpec(memory_space=pltpu.VMEM),
    )(
        x, partials, W1,
        b1.reshape(1, _D), bn1_gamma.reshape(1, _D), bn1_beta.reshape(1, _D),
        W2, b2.reshape(1, _D), gine_eps.reshape(1),
        bn2_gamma.reshape(1, _D), bn2_beta.reshape(1, _D),
    )
    return out
